# Initial kernel scaffold; baseline (speedup 1.0000x reference)
import jax
import jax.numpy as jnp
from jax.experimental import pallas as pl


def kernel(target_deltas, target_scores, output_deltas, output_scores):
    raise NotImplementedError("write your pallas kernel here")


# speedup vs baseline: 1.2147x; 1.2147x over previous
import functools

import jax
import jax.numpy as jnp
from jax import lax
from jax.experimental import pallas as pl
from jax.experimental.pallas import tpu as pltpu
from jax.experimental.pallas import tpu_sc as plsc

N = 16000
C = 81
EPS = 1e-7

NC, NS, L = 2, 16, 16
NW = NC * NS
NPAD = 16384
RPW = NPAD // NW
NBLK = 5
BN = N // NBLK


def _sigmoid(x):
    return 1.0 / (1.0 + jnp.exp(-x))


def _tc_body(ts_ref, os_ref, nl_ref, lab_ref, w_ref, w2_ref, aux_ref, acc_ref):
    i = pl.program_id(0)
    ts = ts_ref[...]
    osv = os_ref[...]
    r = jnp.sum(osv, axis=1)
    p = jnp.sum(ts * osv, axis=1)
    cidx = lax.broadcasted_iota(jnp.int32, (BN, C), 1).astype(jnp.float32)
    labf = jnp.sum(ts * cidx, axis=1)
    lab_ref[pl.ds(i, 1), :] = labf.astype(jnp.int32).reshape(1, BN)
    q = jnp.clip(p / r, EPS, 1.0 - EPS)
    nl_ref[pl.ds(i, 1), :] = (-jnp.log(q)).reshape(1, BN)

    @pl.when(i == 0)
    def _():
        acc_ref[...] = jnp.zeros_like(acc_ref)

    acc_ref[0:1, 0:C] += jnp.sum(ts, axis=0, keepdims=True)

    @pl.when(i == NBLK - 1)
    def _():
        counts = acc_ref[...]
        ntot = jnp.sum(counts)
        npos = ntot - acc_ref[0, 0]
        w = _sigmoid(ntot / jnp.maximum(counts, EPS))
        w2 = _sigmoid(npos / jnp.maximum(counts, EPS))
        lane = lax.broadcasted_iota(jnp.int32, (1, 128), 1)
        w2 = jnp.where(lane == 0, 0.0, w2)
        w_ref[...] = w.reshape(128)
        w2_ref[...] = w2.reshape(128)
        inv_pos = 1.0 / jnp.maximum(EPS, npos)
        aux_ref[...] = jnp.full((16,), inv_pos, jnp.float32)


_tc_call = pl.pallas_call(
    _tc_body,
    grid=(NBLK,),
    in_specs=[
        pl.BlockSpec((BN, C), lambda i: (i, 0)),
        pl.BlockSpec((BN, C), lambda i: (i, 0)),
    ],
    out_specs=[
        pl.BlockSpec((NBLK, BN), lambda i: (0, 0)),
        pl.BlockSpec((NBLK, BN), lambda i: (0, 0)),
        pl.BlockSpec((128,), lambda i: (0,)),
        pl.BlockSpec((128,), lambda i: (0,)),
        pl.BlockSpec((16,), lambda i: (0,)),
    ],
    out_shape=[
        jax.ShapeDtypeStruct((NBLK, BN), jnp.float32),
        jax.ShapeDtypeStruct((NBLK, BN), jnp.int32),
        jax.ShapeDtypeStruct((128,), jnp.float32),
        jax.ShapeDtypeStruct((128,), jnp.float32),
        jax.ShapeDtypeStruct((16,), jnp.float32),
    ],
    scratch_shapes=[pltpu.VMEM((1, 128), jnp.float32)],
)


_sc_mesh = plsc.VectorSubcoreMesh(core_axis_name="c", subcore_axis_name="s")


@functools.partial(
    pl.kernel,
    out_type=jax.ShapeDtypeStruct((NW, 2, 16), jnp.float32),
    mesh=_sc_mesh,
    scratch_types=[
        pltpu.VMEM((4, 128), jnp.int32),
        pltpu.VMEM((4, 128, 16), jnp.float32),
        pltpu.VMEM((4, 128, 16), jnp.float32),
        pltpu.VMEM((RPW,), jnp.int32),
        pltpu.VMEM((RPW,), jnp.float32),
        pltpu.VMEM((128,), jnp.float32),
        pltpu.VMEM((128,), jnp.float32),
        pltpu.VMEM((16,), jnp.float32),
        pltpu.VMEM((2, 16), jnp.float32),
        pltpu.SemaphoreType.DMA,
        pltpu.SemaphoreType.DMA,
    ],
    compiler_params=pltpu.CompilerParams(
        needs_layout_passes=False, use_tc_tiling_on_sc=False
    ),
)
def _sc_call(od_hbm, td_hbm, lab_hbm, nl_hbm, w_hbm, w2_hbm, aux_hbm, out_hbm,
             idx_v, odv, tdv, lab_v, nl_v, w_v, w2_v, aux_v, out_v, sem1, sem2):
    wid = lax.axis_index("s") * NC + lax.axis_index("c")
    base = wid * RPW
    pltpu.sync_copy(lab_hbm.at[pl.ds(base, RPW)], lab_v)
    pltpu.sync_copy(nl_hbm.at[pl.ds(base, RPW)], nl_v)
    pltpu.sync_copy(w_hbm, w_v)
    pltpu.sync_copy(w2_hbm, w2_v)
    pltpu.sync_copy(aux_hbm, aux_v)

    lane = lax.iota(jnp.int32, L)
    for m in range(RPW // L):
        lab16 = lab_v[pl.ds(m * L, L)]
        gn = base + m * L + lane
        idx = jnp.where(gn < N, gn * C + lab16, 0) >> 2
        idx_v[m // 8, pl.ds((m % 8) * L, L)] = idx

    copies = []
    for j in range(4):
        copies.append(pltpu.async_copy(od_hbm.at[idx_v.at[j]], odv.at[j], sem1))
        copies.append(pltpu.async_copy(td_hbm.at[idx_v.at[j]], tdv.at[j], sem2))

    cacc = jnp.zeros((L,), jnp.float32)
    for m in range(RPW // L):
        lab16 = lab_v[pl.ds(m * L, L)]
        wv = plsc.load_gather(w_v, [lab16])
        cacc = cacc + nl_v[pl.ds(m * L, L)] * wv

    for cp in copies:
        cp.wait()

    racc = jnp.zeros((L,), jnp.float32)
    for j in range(4):
        for m in range(32):
            e = m * L + lane
            row = e >> 2
            col = e & 3
            lr = plsc.load_gather(lab_v, [j * 128 + row])
            q = ((base + j * 128 + row) * C + lr) & 3
            col = (q << 2) | col
            o16 = plsc.load_gather(odv.at[j], [row, col])
            t16 = plsc.load_gather(tdv.at[j], [row, col])
            s = plsc.load_gather(w2_v, [lr])
            d = jnp.abs(o16 - t16) * s
            racc = racc + jnp.where(d < 1.0, 0.5 * d * d, d - 0.5)

    out_v[0, :] = cacc * (1.0 / N)
    out_v[1, :] = racc * aux_v[...]
    pltpu.sync_copy(out_v, out_hbm.at[wid])


def kernel(target_deltas, target_scores, output_deltas, output_scores):
    ts2 = target_scores.reshape(N, C)
    os2 = output_scores.reshape(N, C)
    nl, lab, w, w2, aux = _tc_call(ts2, os2)
    labp = jnp.pad(lab.reshape(N), (0, NPAD - N))
    nlp = jnp.pad(nl.reshape(N), (0, NPAD - N))
    od_t = output_deltas.reshape(N * C // 4, 16)
    td_t = target_deltas.reshape(N * C // 4, 16)
    parts = _sc_call(od_t, td_t, labp, nlp, w, w2, aux)
    return jnp.sum(parts)


# speedup vs baseline: 1.8484x; 1.5218x over previous
import functools

import jax
import jax.numpy as jnp
from jax import lax
from jax.experimental import pallas as pl
from jax.experimental.pallas import tpu as pltpu
from jax.experimental.pallas import tpu_sc as plsc

N = 16000
C = 81
C4 = 4 * C
EPS = 1e-7

NC, NS, L = 2, 16, 16
NW = NC * NS
NPAD = 16384
RPW = NPAD // NW
NBLK = 5
BN = N // NBLK


def _sigmoid(x):
    return 1.0 / (1.0 + jnp.exp(-x))


def _stats_body(ts_ref, os_ref, nl_ref, lab_ref, w_ref, w2_ref, aux_ref, wflat_ref, acc_ref):
    i = pl.program_id(0)
    ts = ts_ref[...]
    osv = os_ref[...]
    r = jnp.sum(osv, axis=1)
    p = jnp.sum(ts * osv, axis=1)
    cidx = lax.broadcasted_iota(jnp.int32, (BN, C), 1).astype(jnp.float32)
    labf = jnp.sum(ts * cidx, axis=1)
    lab_ref[pl.ds(i * BN, BN)] = labf.astype(jnp.int32)
    q = jnp.clip(p / r, EPS, 1.0 - EPS)
    nl_ref[pl.ds(i * BN, BN)] = -jnp.log(q)

    @pl.when(i == 0)
    def _():
        acc_ref[...] = jnp.zeros_like(acc_ref)

    acc_ref[0:1, 0:C] += jnp.sum(ts, axis=0, keepdims=True)

    @pl.when(i == NBLK - 1)
    def _():
        lab_ref[pl.ds(N, NPAD - N)] = jnp.zeros((NPAD - N,), jnp.int32)
        nl_ref[pl.ds(N, NPAD - N)] = jnp.zeros((NPAD - N,), jnp.float32)
        counts = acc_ref[...]
        ntot = jnp.sum(counts)
        npos = ntot - acc_ref[0, 0]
        w = _sigmoid(ntot / jnp.maximum(counts, EPS))
        w2 = _sigmoid(npos / jnp.maximum(counts, EPS))
        lane = lax.broadcasted_iota(jnp.int32, (1, 128), 1)
        w2 = jnp.where(lane == 0, 0.0, w2)
        w_ref[...] = w
        w2_ref[...] = w2
        wflat_ref[...] = w.reshape(128)
        inv_pos = 1.0 / jnp.maximum(EPS, npos)
        aux_ref[...] = jnp.full((1, 16), inv_pos, jnp.float32)


_stats_call = pl.pallas_call(
    _stats_body,
    grid=(NBLK,),
    in_specs=[
        pl.BlockSpec((BN, C), lambda i: (i, 0)),
        pl.BlockSpec((BN, C), lambda i: (i, 0)),
    ],
    out_specs=[
        pl.BlockSpec((NPAD,), lambda i: (0,)),
        pl.BlockSpec((NPAD,), lambda i: (0,)),
        pl.BlockSpec((1, 128), lambda i: (0, 0)),
        pl.BlockSpec((1, 128), lambda i: (0, 0)),
        pl.BlockSpec((1, 16), lambda i: (0, 0)),
        pl.BlockSpec((128,), lambda i: (0,)),
    ],
    out_shape=[
        jax.ShapeDtypeStruct((NPAD,), jnp.float32),
        jax.ShapeDtypeStruct((NPAD,), jnp.int32),
        jax.ShapeDtypeStruct((1, 128), jnp.float32),
        jax.ShapeDtypeStruct((1, 128), jnp.float32),
        jax.ShapeDtypeStruct((1, 16), jnp.float32),
        jax.ShapeDtypeStruct((128,), jnp.float32),
    ],
    scratch_shapes=[pltpu.VMEM((1, 128), jnp.float32)],
)


def _reg_body(od_ref, td_ref, ts_ref, w2_ref, aux_ref, out_ref, acc_ref):
    i = pl.program_id(0)
    od = od_ref[...]
    td = td_ref[...]
    ts = ts_ref[...]
    cc4 = lax.broadcasted_iota(jnp.int32, (C, C4), 1) >> 2
    cr = lax.broadcasted_iota(jnp.int32, (C, C4), 0)
    rmat = (cc4 == cr).astype(jnp.float32)
    w2v = w2_ref[...]
    w2rep = jnp.dot(w2v[:, :C], rmat)
    wfull = jnp.dot(ts, rmat) * w2rep
    d = jnp.abs(od - td) * wfull
    sl = jnp.where(d < 1.0, 0.5 * d * d, d - 0.5)

    @pl.when(i == 0)
    def _():
        acc_ref[0, 0] = 0.0

    acc_ref[0, 0] += jnp.sum(sl)

    @pl.when(i == NBLK - 1)
    def _():
        out_ref[0, 0] = acc_ref[0, 0] * aux_ref[0, 0]


_reg_call = pl.pallas_call(
    _reg_body,
    grid=(NBLK,),
    in_specs=[
        pl.BlockSpec((BN, C4), lambda i: (i, 0)),
        pl.BlockSpec((BN, C4), lambda i: (i, 0)),
        pl.BlockSpec((BN, C), lambda i: (i, 0)),
        pl.BlockSpec((1, 128), lambda i: (0, 0)),
        pl.BlockSpec((1, 16), lambda i: (0, 0)),
    ],
    out_specs=pl.BlockSpec(memory_space=pltpu.SMEM),
    out_shape=jax.ShapeDtypeStruct((1, 1), jnp.float32),
    scratch_shapes=[pltpu.SMEM((1, 1), jnp.float32)],
)


_sc_mesh = plsc.VectorSubcoreMesh(core_axis_name="c", subcore_axis_name="s")


@functools.partial(
    pl.kernel,
    out_type=jax.ShapeDtypeStruct((NW, 16), jnp.float32),
    mesh=_sc_mesh,
    scratch_types=[
        pltpu.VMEM((RPW,), jnp.int32),
        pltpu.VMEM((RPW,), jnp.float32),
        pltpu.VMEM((128,), jnp.float32),
        pltpu.VMEM((16,), jnp.float32),
    ],
    compiler_params=pltpu.CompilerParams(
        needs_layout_passes=False, use_tc_tiling_on_sc=False
    ),
)
def _cls_call(lab_hbm, nl_hbm, w_hbm, out_hbm, lab_v, nl_v, w_v, out_v):
    wid = lax.axis_index("s") * NC + lax.axis_index("c")
    base = wid * RPW
    pltpu.sync_copy(lab_hbm.at[pl.ds(base, RPW)], lab_v)
    pltpu.sync_copy(nl_hbm.at[pl.ds(base, RPW)], nl_v)
    pltpu.sync_copy(w_hbm, w_v)

    cacc = jnp.zeros((L,), jnp.float32)
    for m in range(RPW // L):
        lab16 = lab_v[pl.ds(m * L, L)]
        wv = plsc.load_gather(w_v, [lab16])
        cacc = cacc + nl_v[pl.ds(m * L, L)] * wv

    out_v[...] = cacc * (1.0 / N)
    pltpu.sync_copy(out_v, out_hbm.at[wid])


def kernel(target_deltas, target_scores, output_deltas, output_scores):
    ts2 = target_scores.reshape(N, C)
    os2 = output_scores.reshape(N, C)
    nl, lab, w, w2, aux, wflat = _stats_call(ts2, os2)
    od2 = output_deltas.reshape(N, C4)
    td2 = target_deltas.reshape(N, C4)
    reg = _reg_call(od2, td2, ts2, w2, aux)
    cls_parts = _cls_call(lab, nl, wflat)
    return jnp.sum(cls_parts) + reg[0, 0]
